# baseline (speedup 1.0000x reference)
import functools

import jax
import jax.numpy as jnp
from jax.experimental import pallas as pl
from jax.experimental.pallas import tpu as pltpu


def _se_step(x_ref, w1t_ref, w2t_ref, o_ref, y_ref, *, inv_hw, s):
    t = pl.program_id(1)

    @pl.when(t == 0)
    def _gate():
        slab = x_ref[0]
        avg = jnp.sum(slab, axis=0, keepdims=True) * inv_hw
        h = jnp.maximum(
            jnp.dot(avg, w1t_ref[...],
                    preferred_element_type=jnp.float32), 0.0)
        y_ref[...] = jax.nn.sigmoid(
            jnp.dot(h, w2t_ref[...],
                    preferred_element_type=jnp.float32))

    o_ref[0] = x_ref[0, pl.ds(t * s, s), :] * y_ref[...]


def kernel(x_nchw, w1, w2):
    B, C, H, W = x_nchw.shape
    HW = H * W
    Cr = w1.shape[0]
    nt = 8
    s = HW // nt

    x_flat = jnp.transpose(x_nchw, (0, 2, 3, 1)).reshape(B, HW, C)
    w1t = w1.T
    w2t = w2.T

    out_flat = pl.pallas_call(
        functools.partial(_se_step, inv_hw=1.0 / float(HW), s=s),
        out_shape=jax.ShapeDtypeStruct((B, HW, C), x_nchw.dtype),
        grid=(B, nt),
        in_specs=[
            pl.BlockSpec((1, HW, C), lambda b, t: (b, 0, 0)),
            pl.BlockSpec((C, Cr), lambda b, t: (0, 0)),
            pl.BlockSpec((Cr, C), lambda b, t: (0, 0)),
        ],
        out_specs=pl.BlockSpec((1, s, C), lambda b, t: (b, t, 0)),
        scratch_shapes=[
            pltpu.VMEM((1, C), jnp.float32),
        ],
        compiler_params=pltpu.CompilerParams(
            dimension_semantics=("parallel", "arbitrary"),
            vmem_limit_bytes=64 << 20),
    )(x_flat, w1t, w2t)

    return jnp.transpose(out_flat.reshape(B, H, W, C), (0, 3, 1, 2))


# speedup vs baseline: 1.0007x; 1.0007x over previous
import functools

import jax
import jax.numpy as jnp
from jax.experimental import pallas as pl
from jax.experimental.pallas import tpu as pltpu


def _se_step(x_ref, w1t_ref, w2t_ref, o_ref, y_ref, *, inv_hw, s):
    t = pl.program_id(1)

    @pl.when(t == 0)
    def _gate():
        slab = x_ref[0]
        avg = jnp.sum(slab, axis=0, keepdims=True) * inv_hw
        h = jnp.maximum(
            jnp.dot(avg, w1t_ref[...],
                    preferred_element_type=jnp.float32), 0.0)
        y_ref[0] = jax.nn.sigmoid(
            jnp.dot(h, w2t_ref[...],
                    preferred_element_type=jnp.float32))

    o_ref[0] = x_ref[0, pl.ds(t * s, s), :] * y_ref[0]


def kernel(x_nchw, w1, w2):
    B, C, H, W = x_nchw.shape
    HW = H * W
    Cr = w1.shape[0]
    nt = 8
    s = HW // nt

    x_flat = jnp.transpose(x_nchw, (0, 2, 3, 1)).reshape(B, HW, C)
    w1t = w1.T
    w2t = w2.T

    out_flat, _ = pl.pallas_call(
        functools.partial(_se_step, inv_hw=1.0 / float(HW), s=s),
        out_shape=[
            jax.ShapeDtypeStruct((B, HW, C), x_nchw.dtype),
            jax.ShapeDtypeStruct((B, 1, C), jnp.float32),
        ],
        grid=(B, nt),
        in_specs=[
            pl.BlockSpec((1, HW, C), lambda b, t: (b, 0, 0)),
            pl.BlockSpec((C, Cr), lambda b, t: (0, 0)),
            pl.BlockSpec((Cr, C), lambda b, t: (0, 0)),
        ],
        out_specs=[
            pl.BlockSpec((1, s, C), lambda b, t: (b, t, 0)),
            pl.BlockSpec((1, 1, C), lambda b, t: (b, 0, 0)),
        ],
        compiler_params=pltpu.CompilerParams(
            dimension_semantics=("parallel", "arbitrary"),
            vmem_limit_bytes=64 << 20),
    )(x_flat, w1t, w2t)

    return jnp.transpose(out_flat.reshape(B, H, W, C), (0, 3, 1, 2))


# speedup vs baseline: 1.1048x; 1.1040x over previous
import functools

import jax
import jax.numpy as jnp
from jax.experimental import pallas as pl
from jax.experimental.pallas import tpu as pltpu


def _se_step(x_ref, w1t_ref, w2t_ref, o_ref, *, inv_hw, s, nt):
    t = pl.program_id(0) % nt
    slab = x_ref[0]
    avg = jnp.sum(slab, axis=0, keepdims=True) * inv_hw
    h = jnp.maximum(
        jnp.dot(avg, w1t_ref[...], preferred_element_type=jnp.float32), 0.0)
    gate = jax.nn.sigmoid(
        jnp.dot(h, w2t_ref[...], preferred_element_type=jnp.float32))
    o_ref[0] = x_ref[0, pl.ds(t * s, s), :] * gate


def kernel(x_nchw, w1, w2):
    B, C, H, W = x_nchw.shape
    HW = H * W
    Cr = w1.shape[0]
    nt = 4
    s = HW // nt

    x_flat = jnp.transpose(x_nchw, (0, 2, 3, 1)).reshape(B, HW, C)
    w1t = w1.T
    w2t = w2.T

    out_flat = pl.pallas_call(
        functools.partial(_se_step, inv_hw=1.0 / float(HW), s=s, nt=nt),
        out_shape=jax.ShapeDtypeStruct((B, HW, C), x_nchw.dtype),
        grid=(B * nt,),
        in_specs=[
            pl.BlockSpec((1, HW, C), lambda i: (i // nt, 0, 0)),
            pl.BlockSpec((C, Cr), lambda i: (0, 0)),
            pl.BlockSpec((Cr, C), lambda i: (0, 0)),
        ],
        out_specs=pl.BlockSpec((1, s, C), lambda i: (i // nt, i % nt, 0)),
        compiler_params=pltpu.CompilerParams(
            dimension_semantics=("parallel",),
            vmem_limit_bytes=64 << 20),
    )(x_flat, w1t, w2t)

    return jnp.transpose(out_flat.reshape(B, H, W, C), (0, 3, 1, 2))


# speedup vs baseline: 1.1120x; 1.0065x over previous
import functools

import jax
import jax.numpy as jnp
from jax.experimental import pallas as pl
from jax.experimental.pallas import tpu as pltpu


def _se_step(x_ref, w1t_ref, w2t_ref, o_ref, *, inv_hw, s, nt):
    t = pl.program_id(1)
    slab = x_ref[0]
    avg = jnp.sum(slab, axis=0, keepdims=True) * inv_hw
    h = jnp.maximum(
        jnp.dot(avg, w1t_ref[...], preferred_element_type=jnp.float32), 0.0)
    gate = jax.nn.sigmoid(
        jnp.dot(h, w2t_ref[...], preferred_element_type=jnp.float32))
    o_ref[0] = x_ref[0, pl.ds(t * s, s), :] * gate


def kernel(x_nchw, w1, w2):
    B, C, H, W = x_nchw.shape
    HW = H * W
    Cr = w1.shape[0]
    nt = 4
    s = HW // nt

    x_flat = jnp.transpose(x_nchw, (0, 2, 3, 1)).reshape(B, HW, C)
    w1t = w1.T
    w2t = w2.T

    out_flat = pl.pallas_call(
        functools.partial(_se_step, inv_hw=1.0 / float(HW), s=s, nt=nt),
        out_shape=jax.ShapeDtypeStruct((B, HW, C), x_nchw.dtype),
        grid=(B, nt),
        in_specs=[
            pl.BlockSpec((1, HW, C), lambda b, t: (b, 0, 0)),
            pl.BlockSpec((C, Cr), lambda b, t: (0, 0)),
            pl.BlockSpec((Cr, C), lambda b, t: (0, 0)),
        ],
        out_specs=pl.BlockSpec((1, s, C), lambda b, t: (b, t, 0)),
        compiler_params=pltpu.CompilerParams(
            dimension_semantics=("parallel", "parallel"),
            vmem_limit_bytes=64 << 20),
    )(x_flat, w1t, w2t)

    return jnp.transpose(out_flat.reshape(B, H, W, C), (0, 3, 1, 2))


# speedup vs baseline: 2.5206x; 2.2667x over previous
import functools

import jax
import jax.numpy as jnp
from jax.experimental import pallas as pl
from jax.experimental.pallas import tpu as pltpu


def _se_step(x_ref, w1t_ref, w2t_ref, o_ref, *, inv_hw):
    slab = x_ref[0]
    avg = jnp.sum(slab, axis=0, keepdims=True) * inv_hw
    h = jnp.maximum(
        jnp.dot(avg, w1t_ref[...], preferred_element_type=jnp.float32), 0.0)
    gate = jax.nn.sigmoid(
        jnp.dot(h, w2t_ref[...], preferred_element_type=jnp.float32))
    o_ref[0] = slab * gate


def kernel(x_nchw, w1, w2):
    B, C, H, W = x_nchw.shape
    HW = H * W
    Cr = w1.shape[0]

    x_flat = jnp.transpose(x_nchw, (0, 2, 3, 1)).reshape(B, HW, C)
    w1t = w1.T
    w2t = w2.T

    out_flat = pl.pallas_call(
        functools.partial(_se_step, inv_hw=1.0 / float(HW)),
        out_shape=jax.ShapeDtypeStruct((B, HW, C), x_nchw.dtype),
        grid=(B,),
        in_specs=[
            pl.BlockSpec((1, HW, C), lambda b: (b, 0, 0)),
            pl.BlockSpec((C, Cr), lambda b: (0, 0)),
            pl.BlockSpec((Cr, C), lambda b: (0, 0)),
        ],
        out_specs=pl.BlockSpec((1, HW, C), lambda b: (b, 0, 0)),
        compiler_params=pltpu.CompilerParams(
            dimension_semantics=("parallel",),
            vmem_limit_bytes=64 << 20),
    )(x_flat, w1t, w2t)

    return jnp.transpose(out_flat.reshape(B, H, W, C), (0, 3, 1, 2))


# speedup vs baseline: 2.6984x; 1.0706x over previous
import functools

import jax
import jax.numpy as jnp
from jax.experimental import pallas as pl
from jax.experimental.pallas import tpu as pltpu


def _se_step(x_ref, w1t_ref, w2t_ref, o_ref, *, inv_hw):
    slabs = x_ref[...]
    avg = jnp.sum(slabs, axis=1) * inv_hw
    h = jnp.maximum(
        jnp.dot(avg, w1t_ref[...], preferred_element_type=jnp.float32), 0.0)
    gate = jax.nn.sigmoid(
        jnp.dot(h, w2t_ref[...], preferred_element_type=jnp.float32))
    o_ref[...] = slabs * gate[:, None, :]


def kernel(x_nchw, w1, w2):
    B, C, H, W = x_nchw.shape
    HW = H * W
    Cr = w1.shape[0]

    x_flat = jnp.transpose(x_nchw, (0, 2, 3, 1)).reshape(B, HW, C)
    w1t = w1.T
    w2t = w2.T

    out_flat = pl.pallas_call(
        functools.partial(_se_step, inv_hw=1.0 / float(HW)),
        out_shape=jax.ShapeDtypeStruct((B, HW, C), x_nchw.dtype),
        grid=(B // 2,),
        in_specs=[
            pl.BlockSpec((2, HW, C), lambda b: (b, 0, 0)),
            pl.BlockSpec((C, Cr), lambda b: (0, 0)),
            pl.BlockSpec((Cr, C), lambda b: (0, 0)),
        ],
        out_specs=pl.BlockSpec((2, HW, C), lambda b: (b, 0, 0)),
        compiler_params=pltpu.CompilerParams(
            dimension_semantics=("parallel",),
            vmem_limit_bytes=64 << 20),
    )(x_flat, w1t, w2t)

    return jnp.transpose(out_flat.reshape(B, H, W, C), (0, 3, 1, 2))


# speedup vs baseline: 2.7439x; 1.0169x over previous
import functools

import jax
import jax.numpy as jnp
from jax.experimental import pallas as pl
from jax.experimental.pallas import tpu as pltpu


def _se_step(x_ref, w1t_ref, w2t_ref, o_ref, *, inv_hw):
    slabs = x_ref[...]
    avg = jnp.sum(slabs, axis=1) * inv_hw
    h = jnp.maximum(
        jnp.dot(avg, w1t_ref[...], preferred_element_type=jnp.float32), 0.0)
    gate = jax.nn.sigmoid(
        jnp.dot(h, w2t_ref[...], preferred_element_type=jnp.float32))
    o_ref[...] = slabs * gate[:, None, :]


def kernel(x_nchw, w1, w2):
    B, C, H, W = x_nchw.shape
    HW = H * W
    Cr = w1.shape[0]

    x_flat = jnp.transpose(x_nchw, (0, 2, 3, 1)).reshape(B, HW, C)
    w1t = w1.T
    w2t = w2.T

    out_flat = pl.pallas_call(
        functools.partial(_se_step, inv_hw=1.0 / float(HW)),
        out_shape=jax.ShapeDtypeStruct((B, HW, C), x_nchw.dtype),
        grid=(B // 4,),
        in_specs=[
            pl.BlockSpec((4, HW, C), lambda b: (b, 0, 0)),
            pl.BlockSpec((C, Cr), lambda b: (0, 0)),
            pl.BlockSpec((Cr, C), lambda b: (0, 0)),
        ],
        out_specs=pl.BlockSpec((4, HW, C), lambda b: (b, 0, 0)),
        compiler_params=pltpu.CompilerParams(
            dimension_semantics=("parallel",),
            vmem_limit_bytes=64 << 20),
    )(x_flat, w1t, w2t)

    return jnp.transpose(out_flat.reshape(B, H, W, C), (0, 3, 1, 2))
